# Initial kernel scaffold; baseline (speedup 1.0000x reference)
import jax
import jax.numpy as jnp
from jax.experimental import pallas as pl


def kernel(x, table):
    raise NotImplementedError("write your pallas kernel here")


# speedup vs baseline: 4.7274x; 4.7274x over previous
import functools
import math

import jax
import jax.numpy as jnp
from jax import lax
from jax.experimental import pallas as pl
from jax.experimental.pallas import tpu as pltpu
from jax.experimental.pallas import tpu_sc as plsc

_NC = 2
_NS = 16
_NW = _NC * _NS
_C = 128
_L = 16


def _emb_body(n_chunks, d, scale, x_hbm, table_hbm, out_hbm, idx_v, rows_v, sem):
    wid = lax.axis_index("s") * _NC + lax.axis_index("c")
    base = wid * (n_chunks * _C)
    pltpu.sync_copy(x_hbm.at[wid], idx_v)

    def do_chunk(j, carry):
        pltpu.async_copy(table_hbm.at[idx_v.at[j]], rows_v, sem).wait()

        def scale_row(r, cr):
            for k in range(d // _L):
                sl = pl.ds(k * _L, _L)
                rows_v[r, sl] = rows_v[r, sl] * scale
            return cr

        lax.fori_loop(0, _C, scale_row, 0)
        pltpu.sync_copy(rows_v, out_hbm.at[pl.ds(base + j * _C, _C)])
        return carry

    lax.fori_loop(0, n_chunks, do_chunk, 0)


def kernel(x, table):
    b0, s0 = x.shape
    _, d = table.shape
    b = b0 * s0
    assert b % (_NW * _C) == 0 and d % _L == 0
    n_chunks = b // (_NW * _C)
    scale = math.sqrt(d)
    xr = x.reshape(_NW, n_chunks, _C).astype(jnp.int32)

    mesh = plsc.VectorSubcoreMesh(
        core_axis_name="c", subcore_axis_name="s",
        num_cores=_NC, num_subcores=_NS)
    run = pl.kernel(
        functools.partial(_emb_body, n_chunks, d, scale),
        out_type=jax.ShapeDtypeStruct((b, d), jnp.float32),
        mesh=mesh,
        scratch_types=[
            pltpu.VMEM((n_chunks, _C), jnp.int32),
            pltpu.VMEM((_C, d), jnp.float32),
            pltpu.SemaphoreType.DMA,
        ],
    )
    out = run(xr, table)
    return out.reshape(b0, s0, d)


# speedup vs baseline: 7.3167x; 1.5477x over previous
import functools
import math

import jax
import jax.numpy as jnp
from jax import lax
from jax.experimental import pallas as pl
from jax.experimental.pallas import tpu as pltpu
from jax.experimental.pallas import tpu_sc as plsc

_NC = 2
_NS = 16
_NW = _NC * _NS
_C = 80
_L = 16
_NBUF = 4


def _emb_body(n_chunks, d, scale, x_hbm, table_hbm, out_hbm, idx_v,
              bufs, gsems, ssems):
    wid = lax.axis_index("s") * _NC + lax.axis_index("c")
    base = wid * (n_chunks * _C)
    pltpu.sync_copy(x_hbm.at[wid], idx_v)

    def gather(j, b):
        return pltpu.async_copy(table_hbm.at[idx_v.at[j]], bufs[b], gsems[b])

    def gather_wait(j, b):
        pltpu.make_async_copy(table_hbm.at[idx_v.at[j]], bufs[b],
                              gsems[b]).wait()

    def out_slice(j):
        return out_hbm.at[pl.ds(base + j * _C, _C)]

    def scatter(j, b):
        return pltpu.async_copy(bufs[b], out_slice(j), ssems[b])

    def scatter_wait(j, b):
        pltpu.make_async_copy(bufs[b], out_slice(j), ssems[b]).wait()

    def scale_buf(b):
        buf = bufs[b]

        def scale_row(r, cr):
            for k in range(d // _L):
                sl = pl.ds(k * _L, _L)
                buf[r, sl] = buf[r, sl] * scale
            return cr

        lax.fori_loop(0, _C, scale_row, 0)

    gather(0, 0)
    gather(1, 1)

    def quad(g, carry):
        for b in range(_NBUF):
            j = g * _NBUF + b
            nb = (b + 2) % _NBUF
            gather_wait(j, b)
            scale_buf(b)
            scatter(j, b)

            @pl.when(j + 2 < n_chunks)
            def _():
                @pl.when(j >= 2)
                def _():
                    scatter_wait(j - 2, nb)
                gather(j + 2, nb)

        return carry

    lax.fori_loop(0, n_chunks // _NBUF, quad, 0)

    for b in range(_NBUF):
        scatter_wait(n_chunks - _NBUF + b, b)


def kernel(x, table):
    b0, s0 = x.shape
    _, d = table.shape
    b = b0 * s0
    assert b % (_NW * _C) == 0 and d % _L == 0
    n_chunks = b // (_NW * _C)
    assert n_chunks % _NBUF == 0 and n_chunks >= 2 * _NBUF
    scale = math.sqrt(d)
    xr = x.reshape(_NW, n_chunks, _C).astype(jnp.int32)

    mesh = plsc.VectorSubcoreMesh(
        core_axis_name="c", subcore_axis_name="s",
        num_cores=_NC, num_subcores=_NS)
    run = pl.kernel(
        functools.partial(_emb_body, n_chunks, d, scale),
        out_type=jax.ShapeDtypeStruct((b, d), jnp.float32),
        mesh=mesh,
        scratch_types=[
            pltpu.VMEM((n_chunks, _C), jnp.int32),
            [pltpu.VMEM((_C, d), jnp.float32) for _ in range(_NBUF)],
            [pltpu.SemaphoreType.DMA for _ in range(_NBUF)],
            [pltpu.SemaphoreType.DMA for _ in range(_NBUF)],
        ],
    )
    out = run(xr, table)
    return out.reshape(b0, s0, d)
